# initial kernel scaffold (unmeasured)
import jax
import jax.numpy as jnp
from jax import lax
from jax.experimental import pallas as pl
from jax.experimental.pallas import tpu as pltpu

N_DEV = 4
N_HOPS = 2 * (N_DEV - 1)


def kernel(x, w_mat):
    m, k_per = x.shape
    _, n = w_mat.shape
    ch = m // N_DEV

    def body(x_ref, w_ref, out_ref, comm_ref, send_sems, recv_sems, credit_sem):
        my = lax.axis_index("i")
        left = lax.rem(my + N_DEV - 1, N_DEV)
        right = lax.rem(my + 1, N_DEV)

        barrier_sem = pltpu.get_barrier_semaphore()
        for nbr in (left, right):
            pl.semaphore_signal(
                barrier_sem, inc=1,
                device_id=(nbr,), device_id_type=pl.DeviceIdType.MESH,
            )
        pl.semaphore_wait(barrier_sem, 2)

        def rows(c):
            return pl.ds(c * ch, ch)

        def partial(c):
            return jnp.dot(
                x_ref[rows(c), :], w_ref[:, :],
                preferred_element_type=jnp.float32,
            )

        def hop(h, c_send, c_recv, add_partial):
            slot = h % 2
            if h >= 2:
                pl.semaphore_wait(credit_sem, 1)
            rdma = pltpu.make_async_remote_copy(
                src_ref=out_ref.at[rows(c_send)],
                dst_ref=comm_ref.at[slot],
                send_sem=send_sems.at[h],
                recv_sem=recv_sems.at[h],
                device_id=(right,),
                device_id_type=pl.DeviceIdType.MESH,
            )
            rdma.start()
            rdma.wait()
            if add_partial:
                out_ref[rows(c_recv), :] = partial(c_recv) + comm_ref[slot]
            else:
                out_ref[rows(c_recv), :] = comm_ref[slot]
            if h <= N_HOPS - 3:
                pl.semaphore_signal(
                    credit_sem, inc=1,
                    device_id=(left,), device_id_type=pl.DeviceIdType.MESH,
                )

        out_ref[rows(my), :] = partial(my)
        for s in range(N_DEV - 1):
            c_send = lax.rem(my - s + N_DEV, N_DEV)
            c_recv = lax.rem(my - s - 1 + N_DEV, N_DEV)
            hop(s, c_send, c_recv, add_partial=True)

        for t in range(N_DEV - 1):
            c_send = lax.rem(my + 1 - t + N_DEV, N_DEV)
            c_recv = lax.rem(my - t + N_DEV, N_DEV)
            hop(N_DEV - 1 + t, c_send, c_recv, add_partial=False)

        y = out_ref[:, :]
        amax = jnp.max(jnp.abs(y))
        scale = amax / 127.0
        q = jnp.clip(jnp.round(y / scale), -127.0, 127.0)
        out_ref[:, :] = q * scale

    return pl.pallas_call(
        body,
        out_shape=jax.ShapeDtypeStruct((m, n), jnp.float32),
        in_specs=[
            pl.BlockSpec(memory_space=pltpu.VMEM),
            pl.BlockSpec(memory_space=pltpu.VMEM),
        ],
        out_specs=pl.BlockSpec(memory_space=pltpu.VMEM),
        scratch_shapes=[
            pltpu.VMEM((2, ch, n), jnp.float32),
            pltpu.SemaphoreType.DMA((N_HOPS,)),
            pltpu.SemaphoreType.DMA((N_HOPS,)),
            pltpu.SemaphoreType.REGULAR,
        ],
        compiler_params=pltpu.CompilerParams(collective_id=0),
    )(x, w_mat)


# baseline (device time: 478158 ns/iter reference)
import jax
import jax.numpy as jnp
from jax import lax
from jax.experimental import pallas as pl
from jax.experimental.pallas import tpu as pltpu

N_DEV = 4


def kernel(x, w_mat):
    m, k_per = x.shape
    _, n = w_mat.shape
    ch = m // N_DEV

    x = x.astype(jnp.bfloat16)
    w_mat = w_mat.astype(jnp.bfloat16)

    def body(x_ref, w_ref, out_ref, acc_ref, amax_ref,
             send_sems, recv_sems, amax_send_sems, amax_recv_sems,
             credit_sem):
        my = lax.axis_index("i")
        left = lax.rem(my + N_DEV - 1, N_DEV)
        right = lax.rem(my + 1, N_DEV)

        barrier_sem = pltpu.get_barrier_semaphore()
        for nbr in (left, right):
            pl.semaphore_signal(
                barrier_sem, inc=1,
                device_id=(nbr,), device_id_type=pl.DeviceIdType.MESH,
            )
        pl.semaphore_wait(barrier_sem, 2)

        def rows(c):
            return pl.ds(c * ch, ch)

        SUB = 256

        def add_partial(slot, c, add):
            for r0 in range(0, ch, SUB):
                p = jnp.dot(
                    x_ref[pl.ds(c * ch + r0, SUB), :], w_ref[:, :],
                    preferred_element_type=jnp.float32,
                )
                sl = pl.ds(r0, SUB)
                if add:
                    acc_ref[slot, sl, :] = acc_ref[slot, sl, :] + p
                else:
                    acc_ref[slot, sl, :] = p

        add_partial(1, my, add=False)
        c_own = lax.rem(my + 1, N_DEV)
        for s in range(N_DEV - 1):
            src_slot = (s + 1) % 2
            dst_slot = s % 2
            if s >= 1:
                pl.semaphore_wait(credit_sem, 1)
            rdma = pltpu.make_async_remote_copy(
                src_ref=acc_ref.at[src_slot],
                dst_ref=acc_ref.at[dst_slot],
                send_sem=send_sems.at[s],
                recv_sem=recv_sems.at[s],
                device_id=(right,),
                device_id_type=pl.DeviceIdType.MESH,
            )
            rdma.start()
            rdma.wait()
            if s < N_DEV - 2:
                pl.semaphore_signal(
                    credit_sem, inc=1,
                    device_id=(left,), device_id_type=pl.DeviceIdType.MESH,
                )
                c_recv = lax.rem(my - s - 1 + N_DEV, N_DEV)
                add_partial(dst_slot, c_recv, add=True)
            else:
                add_partial(dst_slot, c_own, add=True)

        amax_loc = jnp.max(jnp.abs(acc_ref[0, pl.ds(0, SUB), :]))
        for r0 in range(SUB, ch, SUB):
            amax_loc = jnp.maximum(
                amax_loc, jnp.max(jnp.abs(acc_ref[0, pl.ds(r0, SUB), :]))
            )

        amax_ref[my] = jnp.full((8, 128), amax_loc, dtype=jnp.float32)
        for t in range(N_DEV - 1):
            slot = lax.rem(my - t + N_DEV, N_DEV)
            rdma = pltpu.make_async_remote_copy(
                src_ref=amax_ref.at[slot],
                dst_ref=amax_ref.at[slot],
                send_sem=amax_send_sems.at[t],
                recv_sem=amax_recv_sems.at[t],
                device_id=(right,),
                device_id_type=pl.DeviceIdType.MESH,
            )
            rdma.start()
            rdma.wait()
        amax = jnp.max(amax_ref[...])

        scale = amax / 127.0
        inv_scale = 1.0 / scale
        for r0 in range(0, ch, SUB):
            yq = jnp.clip(
                jnp.round(acc_ref[0, pl.ds(r0, SUB), :] * inv_scale),
                -127.0, 127.0,
            )
            out_ref[pl.ds(c_own * ch + r0, SUB), :] = yq.astype(jnp.bfloat16)
        for t in range(N_DEV - 1):
            c_send = lax.rem(my + 1 - t + N_DEV, N_DEV)
            rdma = pltpu.make_async_remote_copy(
                src_ref=out_ref.at[rows(c_send)],
                dst_ref=out_ref.at[rows(c_send)],
                send_sem=send_sems.at[N_DEV - 1 + t],
                recv_sem=recv_sems.at[N_DEV - 1 + t],
                device_id=(right,),
                device_id_type=pl.DeviceIdType.MESH,
            )
            rdma.start()
            rdma.wait()

        for r0 in range(0, m, SUB):
            sl = pl.ds(r0, SUB)
            out_ref[sl, :] = (
                out_ref[sl, :].astype(jnp.float32) * scale
            ).astype(jnp.bfloat16)

    return pl.pallas_call(
        body,
        out_shape=jax.ShapeDtypeStruct((m, n), jnp.bfloat16),
        in_specs=[
            pl.BlockSpec(memory_space=pltpu.VMEM),
            pl.BlockSpec(memory_space=pltpu.VMEM),
        ],
        out_specs=pl.BlockSpec(memory_space=pltpu.VMEM),
        scratch_shapes=[
            pltpu.VMEM((2, ch, n), jnp.float32),
            pltpu.VMEM((N_DEV, 8, 128), jnp.float32),
            pltpu.SemaphoreType.DMA((2 * (N_DEV - 1),)),
            pltpu.SemaphoreType.DMA((2 * (N_DEV - 1),)),
            pltpu.SemaphoreType.DMA((N_DEV - 1,)),
            pltpu.SemaphoreType.DMA((N_DEV - 1,)),
            pltpu.SemaphoreType.REGULAR,
        ],
        compiler_params=pltpu.CompilerParams(
            collective_id=0,
            vmem_limit_bytes=36 * 1024 * 1024,
        ),
    )(x, w_mat)


# device time: 275967 ns/iter; 1.7327x vs baseline; 1.7327x over previous
import jax
import jax.numpy as jnp
from jax import lax
from jax.experimental import pallas as pl
from jax.experimental.pallas import tpu as pltpu

N_DEV = 4


def kernel(x, w_mat):
    m, k_per = x.shape
    _, n = w_mat.shape
    ch = m // N_DEV
    nh = n // 2

    x = x.astype(jnp.bfloat16)
    w_mat = w_mat.astype(jnp.bfloat16)

    def body(x_ref, w_ref, out_ref, acc_cw, acc_ccw, amax_ref,
             send_cw, recv_cw, send_ccw, recv_ccw,
             amax_send_sems, amax_recv_sems, credit_cw, credit_ccw):
        my = lax.axis_index("i")
        left = lax.rem(my + N_DEV - 1, N_DEV)
        right = lax.rem(my + 1, N_DEV)

        barrier_sem = pltpu.get_barrier_semaphore()
        for nbr in (left, right):
            pl.semaphore_signal(
                barrier_sem, inc=1,
                device_id=(nbr,), device_id_type=pl.DeviceIdType.MESH,
            )
        pl.semaphore_wait(barrier_sem, 2)

        SUB = 256

        def add_partial(acc, slot, c, col0, add):
            for r0 in range(0, ch, SUB):
                p = jnp.dot(
                    x_ref[pl.ds(c * ch + r0, SUB), :],
                    w_ref[:, pl.ds(col0, nh)],
                    preferred_element_type=jnp.float32,
                )
                sl = pl.ds(r0, SUB)
                if add:
                    acc[slot, sl, :] = acc[slot, sl, :] + p
                else:
                    acc[slot, sl, :] = p

        c_own_cw = lax.rem(my + 1, N_DEV)
        c_own_ccw = lax.rem(my + N_DEV - 1, N_DEV)
        add_partial(acc_cw, 1, my, 0, add=False)
        add_partial(acc_ccw, 1, my, nh, add=False)
        for s in range(N_DEV - 1):
            src_slot = (s + 1) % 2
            dst_slot = s % 2
            if s >= 1:
                pl.semaphore_wait(credit_cw, 1)
                pl.semaphore_wait(credit_ccw, 1)
            rdma_cw = pltpu.make_async_remote_copy(
                src_ref=acc_cw.at[src_slot],
                dst_ref=acc_cw.at[dst_slot],
                send_sem=send_cw.at[s],
                recv_sem=recv_cw.at[s],
                device_id=(right,),
                device_id_type=pl.DeviceIdType.MESH,
            )
            rdma_ccw = pltpu.make_async_remote_copy(
                src_ref=acc_ccw.at[src_slot],
                dst_ref=acc_ccw.at[dst_slot],
                send_sem=send_ccw.at[s],
                recv_sem=recv_ccw.at[s],
                device_id=(left,),
                device_id_type=pl.DeviceIdType.MESH,
            )
            rdma_cw.start()
            rdma_ccw.start()
            rdma_cw.wait()
            rdma_ccw.wait()
            if s < N_DEV - 2:
                pl.semaphore_signal(
                    credit_cw, inc=1,
                    device_id=(left,), device_id_type=pl.DeviceIdType.MESH,
                )
                pl.semaphore_signal(
                    credit_ccw, inc=1,
                    device_id=(right,), device_id_type=pl.DeviceIdType.MESH,
                )
                c_recv_cw = lax.rem(my - s - 1 + N_DEV, N_DEV)
                c_recv_ccw = lax.rem(my + s + 1, N_DEV)
                add_partial(acc_cw, dst_slot, c_recv_cw, 0, add=True)
                add_partial(acc_ccw, dst_slot, c_recv_ccw, nh, add=True)
            else:
                add_partial(acc_cw, dst_slot, c_own_cw, 0, add=True)
                add_partial(acc_ccw, dst_slot, c_own_ccw, nh, add=True)

        amax_loc = jnp.max(jnp.abs(acc_cw[0, pl.ds(0, SUB), :]))
        for r0 in range(SUB, ch, SUB):
            amax_loc = jnp.maximum(
                amax_loc, jnp.max(jnp.abs(acc_cw[0, pl.ds(r0, SUB), :]))
            )
        for r0 in range(0, ch, SUB):
            amax_loc = jnp.maximum(
                amax_loc, jnp.max(jnp.abs(acc_ccw[0, pl.ds(r0, SUB), :]))
            )

        amax_ref[my] = jnp.full((8, 128), amax_loc, dtype=jnp.float32)
        for t in range(N_DEV - 1):
            slot = lax.rem(my - t + N_DEV, N_DEV)
            rdma = pltpu.make_async_remote_copy(
                src_ref=amax_ref.at[slot],
                dst_ref=amax_ref.at[slot],
                send_sem=amax_send_sems.at[t],
                recv_sem=amax_recv_sems.at[t],
                device_id=(right,),
                device_id_type=pl.DeviceIdType.MESH,
            )
            rdma.start()
            rdma.wait()
        amax = jnp.max(amax_ref[...])

        scale = amax / 127.0
        inv_scale = 1.0 / scale
        for r0 in range(0, ch, SUB):
            yq = jnp.clip(
                jnp.round(acc_cw[0, pl.ds(r0, SUB), :] * inv_scale),
                -127.0, 127.0,
            )
            out_ref[pl.ds(c_own_cw * ch + r0, SUB), pl.ds(0, nh)] = (
                yq.astype(jnp.bfloat16)
            )
            yq = jnp.clip(
                jnp.round(acc_ccw[0, pl.ds(r0, SUB), :] * inv_scale),
                -127.0, 127.0,
            )
            out_ref[pl.ds(c_own_ccw * ch + r0, SUB), pl.ds(nh, nh)] = (
                yq.astype(jnp.bfloat16)
            )
        for t in range(N_DEV - 1):
            c_cw = lax.rem(my + 1 - t + N_DEV, N_DEV)
            c_ccw = lax.rem(my - 1 + t + N_DEV, N_DEV)
            rdma_cw = pltpu.make_async_remote_copy(
                src_ref=out_ref.at[pl.ds(c_cw * ch, ch), pl.ds(0, nh)],
                dst_ref=out_ref.at[pl.ds(c_cw * ch, ch), pl.ds(0, nh)],
                send_sem=send_cw.at[N_DEV - 1 + t],
                recv_sem=recv_cw.at[N_DEV - 1 + t],
                device_id=(right,),
                device_id_type=pl.DeviceIdType.MESH,
            )
            rdma_ccw = pltpu.make_async_remote_copy(
                src_ref=out_ref.at[pl.ds(c_ccw * ch, ch), pl.ds(nh, nh)],
                dst_ref=out_ref.at[pl.ds(c_ccw * ch, ch), pl.ds(nh, nh)],
                send_sem=send_ccw.at[N_DEV - 1 + t],
                recv_sem=recv_ccw.at[N_DEV - 1 + t],
                device_id=(left,),
                device_id_type=pl.DeviceIdType.MESH,
            )
            rdma_cw.start()
            rdma_ccw.start()
            rdma_cw.wait()
            rdma_ccw.wait()

        for r0 in range(0, m, SUB):
            sl = pl.ds(r0, SUB)
            out_ref[sl, :] = (
                out_ref[sl, :].astype(jnp.float32) * scale
            ).astype(jnp.bfloat16)

    return pl.pallas_call(
        body,
        out_shape=jax.ShapeDtypeStruct((m, n), jnp.bfloat16),
        in_specs=[
            pl.BlockSpec(memory_space=pltpu.VMEM),
            pl.BlockSpec(memory_space=pltpu.VMEM),
        ],
        out_specs=pl.BlockSpec(memory_space=pltpu.VMEM),
        scratch_shapes=[
            pltpu.VMEM((2, ch, n // 2), jnp.float32),
            pltpu.VMEM((2, ch, n // 2), jnp.float32),
            pltpu.VMEM((N_DEV, 8, 128), jnp.float32),
            pltpu.SemaphoreType.DMA((2 * (N_DEV - 1),)),
            pltpu.SemaphoreType.DMA((2 * (N_DEV - 1),)),
            pltpu.SemaphoreType.DMA((2 * (N_DEV - 1),)),
            pltpu.SemaphoreType.DMA((2 * (N_DEV - 1),)),
            pltpu.SemaphoreType.DMA((N_DEV - 1,)),
            pltpu.SemaphoreType.DMA((N_DEV - 1,)),
            pltpu.SemaphoreType.REGULAR,
            pltpu.SemaphoreType.REGULAR,
        ],
        compiler_params=pltpu.CompilerParams(
            collective_id=0,
            vmem_limit_bytes=36 * 1024 * 1024,
        ),
    )(x, w_mat)


# device time: 176426 ns/iter; 2.7102x vs baseline; 1.5642x over previous
import jax
import jax.numpy as jnp
from jax import lax
from jax.experimental import pallas as pl
from jax.experimental.pallas import tpu as pltpu

N_DEV = 4


def kernel(x, w_mat):
    m, k_per = x.shape
    _, n = w_mat.shape
    ch = m // N_DEV
    nh = n // 2

    x = x.astype(jnp.bfloat16)
    w_mat = w_mat.astype(jnp.bfloat16)

    def body(x_ref, w_ref, out_ref, acc_cw, acc_ccw, q_ref, amax_ref,
             send_cw, recv_cw, send_ccw, recv_ccw,
             amax_send_sems, amax_recv_sems, credit_cw, credit_ccw):
        my = lax.axis_index("i")
        left = lax.rem(my + N_DEV - 1, N_DEV)
        right = lax.rem(my + 1, N_DEV)

        barrier_sem = pltpu.get_barrier_semaphore()
        for nbr in (left, right):
            pl.semaphore_signal(
                barrier_sem, inc=1,
                device_id=(nbr,), device_id_type=pl.DeviceIdType.MESH,
            )
        pl.semaphore_wait(barrier_sem, 2)

        SUB = 256

        def add_partial(acc, slot, c, col0, add):
            for r0 in range(0, ch, SUB):
                p = jnp.dot(
                    x_ref[pl.ds(c * ch + r0, SUB), :],
                    w_ref[:, pl.ds(col0, nh)],
                    preferred_element_type=jnp.float32,
                )
                sl = pl.ds(r0, SUB)
                if add:
                    p = acc[slot, sl, :].astype(jnp.float32) + p
                acc[slot, sl, :] = p.astype(jnp.bfloat16)

        c_own_cw = lax.rem(my + 1, N_DEV)
        c_own_ccw = lax.rem(my + N_DEV - 1, N_DEV)
        add_partial(acc_cw, 1, my, 0, add=False)
        add_partial(acc_ccw, 1, my, nh, add=False)
        for s in range(N_DEV - 1):
            src_slot = (s + 1) % 2
            dst_slot = s % 2
            if s >= 1:
                pl.semaphore_wait(credit_cw, 1)
                pl.semaphore_wait(credit_ccw, 1)
            rdma_cw = pltpu.make_async_remote_copy(
                src_ref=acc_cw.at[src_slot],
                dst_ref=acc_cw.at[dst_slot],
                send_sem=send_cw.at[s],
                recv_sem=recv_cw.at[s],
                device_id=(right,),
                device_id_type=pl.DeviceIdType.MESH,
            )
            rdma_ccw = pltpu.make_async_remote_copy(
                src_ref=acc_ccw.at[src_slot],
                dst_ref=acc_ccw.at[dst_slot],
                send_sem=send_ccw.at[s],
                recv_sem=recv_ccw.at[s],
                device_id=(left,),
                device_id_type=pl.DeviceIdType.MESH,
            )
            rdma_cw.start()
            rdma_ccw.start()
            rdma_cw.wait()
            rdma_ccw.wait()
            if s < N_DEV - 2:
                pl.semaphore_signal(
                    credit_cw, inc=1,
                    device_id=(left,), device_id_type=pl.DeviceIdType.MESH,
                )
                pl.semaphore_signal(
                    credit_ccw, inc=1,
                    device_id=(right,), device_id_type=pl.DeviceIdType.MESH,
                )
                c_recv_cw = lax.rem(my - s - 1 + N_DEV, N_DEV)
                c_recv_ccw = lax.rem(my + s + 1, N_DEV)
                add_partial(acc_cw, dst_slot, c_recv_cw, 0, add=True)
                add_partial(acc_ccw, dst_slot, c_recv_ccw, nh, add=True)
            else:
                add_partial(acc_cw, dst_slot, c_own_cw, 0, add=True)
                add_partial(acc_ccw, dst_slot, c_own_ccw, nh, add=True)

        def tile_absmax(acc, r0):
            return jnp.max(
                jnp.abs(acc[0, pl.ds(r0, SUB), :].astype(jnp.float32))
            )

        amax_loc = tile_absmax(acc_cw, 0)
        for r0 in range(SUB, ch, SUB):
            amax_loc = jnp.maximum(amax_loc, tile_absmax(acc_cw, r0))
        for r0 in range(0, ch, SUB):
            amax_loc = jnp.maximum(amax_loc, tile_absmax(acc_ccw, r0))

        amax_ref[my] = jnp.full((8, 128), amax_loc, dtype=jnp.float32)
        for t in range(N_DEV - 1):
            slot = lax.rem(my - t + N_DEV, N_DEV)
            rdma = pltpu.make_async_remote_copy(
                src_ref=amax_ref.at[slot],
                dst_ref=amax_ref.at[slot],
                send_sem=amax_send_sems.at[t],
                recv_sem=amax_recv_sems.at[t],
                device_id=(right,),
                device_id_type=pl.DeviceIdType.MESH,
            )
            rdma.start()
            rdma.wait()
        amax = jnp.max(amax_ref[...])

        scale = amax / 127.0
        inv_scale = 1.0 / scale
        for r0 in range(0, ch, SUB):
            yq = jnp.clip(
                jnp.round(
                    acc_cw[0, pl.ds(r0, SUB), :].astype(jnp.float32)
                    * inv_scale
                ),
                -127.0, 127.0,
            )
            q_ref[pl.ds(c_own_cw * ch + r0, SUB), pl.ds(0, nh)] = (
                yq.astype(jnp.int8)
            )
            yq = jnp.clip(
                jnp.round(
                    acc_ccw[0, pl.ds(r0, SUB), :].astype(jnp.float32)
                    * inv_scale
                ),
                -127.0, 127.0,
            )
            q_ref[pl.ds(c_own_ccw * ch + r0, SUB), pl.ds(nh, nh)] = (
                yq.astype(jnp.int8)
            )
        for t in range(N_DEV - 1):
            c_cw = lax.rem(my + 1 - t + N_DEV, N_DEV)
            c_ccw = lax.rem(my - 1 + t + N_DEV, N_DEV)
            rdma_cw = pltpu.make_async_remote_copy(
                src_ref=q_ref.at[pl.ds(c_cw * ch, ch), pl.ds(0, nh)],
                dst_ref=q_ref.at[pl.ds(c_cw * ch, ch), pl.ds(0, nh)],
                send_sem=send_cw.at[N_DEV - 1 + t],
                recv_sem=recv_cw.at[N_DEV - 1 + t],
                device_id=(right,),
                device_id_type=pl.DeviceIdType.MESH,
            )
            rdma_ccw = pltpu.make_async_remote_copy(
                src_ref=q_ref.at[pl.ds(c_ccw * ch, ch), pl.ds(nh, nh)],
                dst_ref=q_ref.at[pl.ds(c_ccw * ch, ch), pl.ds(nh, nh)],
                send_sem=send_ccw.at[N_DEV - 1 + t],
                recv_sem=recv_ccw.at[N_DEV - 1 + t],
                device_id=(left,),
                device_id_type=pl.DeviceIdType.MESH,
            )
            rdma_cw.start()
            rdma_ccw.start()
            rdma_cw.wait()
            rdma_ccw.wait()

        for r0 in range(0, m, SUB):
            sl = pl.ds(r0, SUB)
            out_ref[sl, :] = (
                q_ref[sl, :].astype(jnp.float32) * scale
            ).astype(jnp.bfloat16)

    return pl.pallas_call(
        body,
        out_shape=jax.ShapeDtypeStruct((m, n), jnp.bfloat16),
        in_specs=[
            pl.BlockSpec(memory_space=pltpu.VMEM),
            pl.BlockSpec(memory_space=pltpu.VMEM),
        ],
        out_specs=pl.BlockSpec(memory_space=pltpu.VMEM),
        scratch_shapes=[
            pltpu.VMEM((2, ch, n // 2), jnp.bfloat16),
            pltpu.VMEM((2, ch, n // 2), jnp.bfloat16),
            pltpu.VMEM((m, n), jnp.int8),
            pltpu.VMEM((N_DEV, 8, 128), jnp.float32),
            pltpu.SemaphoreType.DMA((2 * (N_DEV - 1),)),
            pltpu.SemaphoreType.DMA((2 * (N_DEV - 1),)),
            pltpu.SemaphoreType.DMA((2 * (N_DEV - 1),)),
            pltpu.SemaphoreType.DMA((2 * (N_DEV - 1),)),
            pltpu.SemaphoreType.DMA((N_DEV - 1,)),
            pltpu.SemaphoreType.DMA((N_DEV - 1,)),
            pltpu.SemaphoreType.REGULAR,
            pltpu.SemaphoreType.REGULAR,
        ],
        compiler_params=pltpu.CompilerParams(
            collective_id=0,
            vmem_limit_bytes=36 * 1024 * 1024,
        ),
    )(x, w_mat)


# device time: 156712 ns/iter; 3.0512x vs baseline; 1.1258x over previous
import jax
import jax.numpy as jnp
from jax import lax
from jax.experimental import pallas as pl
from jax.experimental.pallas import tpu as pltpu

N_DEV = 4


def kernel(x, w_mat):
    m, k_per = x.shape
    _, n = w_mat.shape
    ch = m // N_DEV
    nh = n // 2

    x = x.astype(jnp.bfloat16)
    w_mat = w_mat.astype(jnp.bfloat16)

    def body(x_ref, w_ref, out_ref, acc_cw, acc_ccw, q_ref, stage_cw,
             stage_ccw, deq_tiles, amax_ref, send_cw, recv_cw, send_ccw,
             recv_ccw, amax_send_sems, amax_recv_sems, deq_sems,
             credit_cw, credit_ccw):
        my = lax.axis_index("i")
        left = lax.rem(my + N_DEV - 1, N_DEV)
        right = lax.rem(my + 1, N_DEV)

        barrier_sem = pltpu.get_barrier_semaphore()
        for nbr in (left, right):
            pl.semaphore_signal(
                barrier_sem, inc=1,
                device_id=(nbr,), device_id_type=pl.DeviceIdType.MESH,
            )
        pl.semaphore_wait(barrier_sem, 2)

        SUB = 256

        def gemm_into(dst, c, col0):
            for r0 in range(0, ch, SUB):
                p = jnp.dot(
                    x_ref[pl.ds(c * ch + r0, SUB), :],
                    w_ref[:, pl.ds(col0, nh)],
                    preferred_element_type=jnp.float32,
                )
                dst[pl.ds(r0, SUB), :] = p.astype(jnp.bfloat16)

        def add_stage(acc, slot, stage):
            for r0 in range(0, ch, SUB):
                sl = pl.ds(r0, SUB)
                acc[slot, sl, :] = (
                    acc[slot, sl, :].astype(jnp.float32)
                    + stage[sl, :].astype(jnp.float32)
                ).astype(jnp.bfloat16)

        c_own_cw = lax.rem(my + 1, N_DEV)
        c_own_ccw = lax.rem(my + N_DEV - 1, N_DEV)
        gemm_into(acc_cw.at[1], my, 0)
        gemm_into(acc_ccw.at[1], my, nh)
        for s in range(N_DEV - 1):
            src_slot = (s + 1) % 2
            dst_slot = s % 2
            if s >= 1:
                pl.semaphore_wait(credit_cw, 1)
                pl.semaphore_wait(credit_ccw, 1)
            rdma_cw = pltpu.make_async_remote_copy(
                src_ref=acc_cw.at[src_slot],
                dst_ref=acc_cw.at[dst_slot],
                send_sem=send_cw.at[s],
                recv_sem=recv_cw.at[s],
                device_id=(right,),
                device_id_type=pl.DeviceIdType.MESH,
            )
            rdma_ccw = pltpu.make_async_remote_copy(
                src_ref=acc_ccw.at[src_slot],
                dst_ref=acc_ccw.at[dst_slot],
                send_sem=send_ccw.at[s],
                recv_sem=recv_ccw.at[s],
                device_id=(left,),
                device_id_type=pl.DeviceIdType.MESH,
            )
            rdma_cw.start()
            rdma_ccw.start()
            c_recv_cw = lax.rem(my - s - 1 + N_DEV, N_DEV)
            c_recv_ccw = lax.rem(my + s + 1, N_DEV)
            gemm_into(stage_cw, c_recv_cw, 0)
            gemm_into(stage_ccw, c_recv_ccw, nh)
            rdma_cw.wait()
            if s < N_DEV - 2:
                pl.semaphore_signal(
                    credit_cw, inc=1,
                    device_id=(left,), device_id_type=pl.DeviceIdType.MESH,
                )
            add_stage(acc_cw, dst_slot, stage_cw)
            rdma_ccw.wait()
            if s < N_DEV - 2:
                pl.semaphore_signal(
                    credit_ccw, inc=1,
                    device_id=(right,), device_id_type=pl.DeviceIdType.MESH,
                )
            add_stage(acc_ccw, dst_slot, stage_ccw)

        def tile_absmax(acc, r0):
            return jnp.max(
                jnp.abs(acc[0, pl.ds(r0, SUB), :].astype(jnp.float32))
            )

        amax_loc = tile_absmax(acc_cw, 0)
        for r0 in range(SUB, ch, SUB):
            amax_loc = jnp.maximum(amax_loc, tile_absmax(acc_cw, r0))
        for r0 in range(0, ch, SUB):
            amax_loc = jnp.maximum(amax_loc, tile_absmax(acc_ccw, r0))

        amax_ref[my] = jnp.full((8, 128), amax_loc, dtype=jnp.float32)
        amax_rdmas = []
        for t in range(N_DEV - 1):
            tgt = lax.rem(my + 1 + t, N_DEV)
            rdma = pltpu.make_async_remote_copy(
                src_ref=amax_ref.at[my],
                dst_ref=amax_ref.at[my],
                send_sem=amax_send_sems.at[t],
                recv_sem=amax_recv_sems.at[t],
                device_id=(tgt,),
                device_id_type=pl.DeviceIdType.MESH,
            )
            rdma.start()
            amax_rdmas.append(rdma)
        for rdma in amax_rdmas:
            rdma.wait()
        amax = jnp.max(amax_ref[...])

        scale = amax / 127.0
        inv_scale = 1.0 / scale
        for r0 in range(0, ch, SUB):
            yq = jnp.clip(
                jnp.round(
                    acc_cw[0, pl.ds(r0, SUB), :].astype(jnp.float32)
                    * inv_scale
                ),
                -127.0, 127.0,
            )
            q_ref[pl.ds(c_own_cw * ch + r0, SUB), pl.ds(0, nh)] = (
                yq.astype(jnp.int8)
            )
            yq = jnp.clip(
                jnp.round(
                    acc_ccw[0, pl.ds(r0, SUB), :].astype(jnp.float32)
                    * inv_scale
                ),
                -127.0, 127.0,
            )
            q_ref[pl.ds(c_own_ccw * ch + r0, SUB), pl.ds(nh, nh)] = (
                yq.astype(jnp.int8)
            )
        deq_pending = {}

        def dequant_half(c, col0):
            for r0 in range(0, ch, SUB):
                slot = (r0 // SUB) % 2
                if slot in deq_pending:
                    deq_pending.pop(slot).wait()
                sl = pl.ds(c * ch + r0, SUB)
                cs = pl.ds(col0, nh)
                deq_tiles[slot] = (
                    q_ref[sl, cs].astype(jnp.float32) * scale
                ).astype(jnp.bfloat16)
                copy = pltpu.make_async_copy(
                    deq_tiles.at[slot],
                    out_ref.at[sl, cs],
                    deq_sems.at[slot],
                )
                copy.start()
                deq_pending[slot] = copy

        for t in range(N_DEV - 1):
            c_cw = lax.rem(my + 1 - t + N_DEV, N_DEV)
            c_ccw = lax.rem(my - 1 + t + N_DEV, N_DEV)
            rdma_cw = pltpu.make_async_remote_copy(
                src_ref=q_ref.at[pl.ds(c_cw * ch, ch), pl.ds(0, nh)],
                dst_ref=q_ref.at[pl.ds(c_cw * ch, ch), pl.ds(0, nh)],
                send_sem=send_cw.at[N_DEV - 1 + t],
                recv_sem=recv_cw.at[N_DEV - 1 + t],
                device_id=(right,),
                device_id_type=pl.DeviceIdType.MESH,
            )
            rdma_ccw = pltpu.make_async_remote_copy(
                src_ref=q_ref.at[pl.ds(c_ccw * ch, ch), pl.ds(nh, nh)],
                dst_ref=q_ref.at[pl.ds(c_ccw * ch, ch), pl.ds(nh, nh)],
                send_sem=send_ccw.at[N_DEV - 1 + t],
                recv_sem=recv_ccw.at[N_DEV - 1 + t],
                device_id=(left,),
                device_id_type=pl.DeviceIdType.MESH,
            )
            rdma_cw.start()
            rdma_ccw.start()
            dequant_half(c_cw, 0)
            dequant_half(c_ccw, nh)
            rdma_cw.wait()
            rdma_ccw.wait()

        dequant_half(lax.rem(my + 2, N_DEV), 0)
        dequant_half(lax.rem(my + 2, N_DEV), nh)
        for copy in deq_pending.values():
            copy.wait()

    return pl.pallas_call(
        body,
        out_shape=jax.ShapeDtypeStruct((m, n), jnp.bfloat16),
        in_specs=[
            pl.BlockSpec(memory_space=pltpu.VMEM),
            pl.BlockSpec(memory_space=pltpu.VMEM),
        ],
        out_specs=pl.BlockSpec(memory_space=pl.ANY),
        scratch_shapes=[
            pltpu.VMEM((2, ch, n // 2), jnp.bfloat16),
            pltpu.VMEM((2, ch, n // 2), jnp.bfloat16),
            pltpu.VMEM((m, n), jnp.int8),
            pltpu.VMEM((ch, n // 2), jnp.bfloat16),
            pltpu.VMEM((ch, n // 2), jnp.bfloat16),
            pltpu.VMEM((2, 256, n // 2), jnp.bfloat16),
            pltpu.VMEM((N_DEV, 8, 128), jnp.float32),
            pltpu.SemaphoreType.DMA((2 * (N_DEV - 1),)),
            pltpu.SemaphoreType.DMA((2 * (N_DEV - 1),)),
            pltpu.SemaphoreType.DMA((2 * (N_DEV - 1),)),
            pltpu.SemaphoreType.DMA((2 * (N_DEV - 1),)),
            pltpu.SemaphoreType.DMA((N_DEV - 1,)),
            pltpu.SemaphoreType.DMA((N_DEV - 1,)),
            pltpu.SemaphoreType.DMA((2,)),
            pltpu.SemaphoreType.REGULAR,
            pltpu.SemaphoreType.REGULAR,
        ],
        compiler_params=pltpu.CompilerParams(
            collective_id=0,
            vmem_limit_bytes=40 * 1024 * 1024,
        ),
    )(x, w_mat)


# device time: 152906 ns/iter; 3.1271x vs baseline; 1.0249x over previous
import functools

import jax
import jax.numpy as jnp
from jax import lax
from jax.experimental import pallas as pl
from jax.experimental.pallas import tpu as pltpu

N_DEV = 4


def kernel(x, w_mat):
    m, k_per = x.shape
    _, n = w_mat.shape
    ch = m // N_DEV
    nh = n // 2

    x = x.astype(jnp.bfloat16)
    w_mat = w_mat.astype(jnp.bfloat16)

    def body(x_ref, w_ref, out_ref, acc_cw, acc_ccw, q_ref, stage_cw,
             stage_ccw, deq_tiles, amax_ref, send_cw, recv_cw, send_ccw,
             recv_ccw, amax_send_sems, amax_recv_sems, deq_sems,
             credit_cw, credit_ccw):
        my = lax.axis_index("i")
        left = lax.rem(my + N_DEV - 1, N_DEV)
        right = lax.rem(my + 1, N_DEV)

        barrier_sem = pltpu.get_barrier_semaphore()
        for nbr in (left, right):
            pl.semaphore_signal(
                barrier_sem, inc=1,
                device_id=(nbr,), device_id_type=pl.DeviceIdType.MESH,
            )
        pl.semaphore_wait(barrier_sem, 2)

        SUB = 256
        HALF = ch // 2

        def gemm_into(dst, c, col0, r_lo, r_hi):
            for r0 in range(r_lo, r_hi, SUB):
                p = jnp.dot(
                    x_ref[pl.ds(c * ch + r0, SUB), :],
                    w_ref[:, pl.ds(col0, nh)],
                    preferred_element_type=jnp.float32,
                )
                dst[pl.ds(r0, SUB), :] = p.astype(jnp.bfloat16)

        own_maxes = []

        def add_stage(acc, slot, stage, r_lo, r_hi, track):
            for r0 in range(r_lo, r_hi, SUB):
                sl = pl.ds(r0, SUB)
                v = (
                    acc[slot, sl, :].astype(jnp.float32)
                    + stage[sl, :].astype(jnp.float32)
                )
                acc[slot, sl, :] = v.astype(jnp.bfloat16)
                if track:
                    own_maxes.append(jnp.max(jnp.abs(v)))

        def rs_rdma(acc, src_slot, dst_slot, s_sems, r_sems, idx, half,
                    tgt):
            return pltpu.make_async_remote_copy(
                src_ref=acc.at[src_slot, pl.ds(half * HALF, HALF), :],
                dst_ref=acc.at[dst_slot, pl.ds(half * HALF, HALF), :],
                send_sem=s_sems.at[idx],
                recv_sem=r_sems.at[idx],
                device_id=(tgt,),
                device_id_type=pl.DeviceIdType.MESH,
            )

        c_own_cw = lax.rem(my + 1, N_DEV)
        c_own_ccw = lax.rem(my + N_DEV - 1, N_DEV)

        pending = {}
        gemm_into(acc_cw.at[1], my, 0, 0, HALF)
        gemm_into(acc_ccw.at[1], my, nh, 0, HALF)
        for d, acc, s_sems, r_sems, tgt in (
            ("cw", acc_cw, send_cw, recv_cw, right),
            ("ccw", acc_ccw, send_ccw, recv_ccw, left),
        ):
            r = rs_rdma(acc, 1, 0, s_sems, r_sems, 0, 0, tgt)
            r.start()
            pending[d, 0] = r
        gemm_into(acc_cw.at[1], my, 0, HALF, ch)
        gemm_into(acc_ccw.at[1], my, nh, HALF, ch)
        for d, acc, s_sems, r_sems, tgt in (
            ("cw", acc_cw, send_cw, recv_cw, right),
            ("ccw", acc_ccw, send_ccw, recv_ccw, left),
        ):
            r = rs_rdma(acc, 1, 0, s_sems, r_sems, 1, 1, tgt)
            r.start()
            pending[d, 1] = r

        for s in range(N_DEV - 1):
            src_slot = (s + 1) % 2
            dst_slot = s % 2
            if s >= 1:
                pl.semaphore_wait(credit_cw, 1)
                pl.semaphore_wait(credit_ccw, 1)
                for sub in (0, 1):
                    r = rs_rdma(acc_cw, src_slot, dst_slot, send_cw,
                                recv_cw, 2 * s + sub, sub, right)
                    r.start()
                    pending["cw", sub] = r
                    r = rs_rdma(acc_ccw, src_slot, dst_slot, send_ccw,
                                recv_ccw, 2 * s + sub, sub, left)
                    r.start()
                    pending["ccw", sub] = r
            c_recv_cw = lax.rem(my - s - 1 + N_DEV, N_DEV)
            c_recv_ccw = lax.rem(my + s + 1, N_DEV)
            gemm_into(stage_cw, c_recv_cw, 0, 0, ch)
            gemm_into(stage_ccw, c_recv_ccw, nh, 0, ch)
            track = s == N_DEV - 2
            pending["cw", 0].wait()
            add_stage(acc_cw, dst_slot, stage_cw, 0, HALF, track)
            pending["ccw", 0].wait()
            add_stage(acc_ccw, dst_slot, stage_ccw, 0, HALF, track)
            pending["cw", 1].wait()
            if s < N_DEV - 2:
                pl.semaphore_signal(
                    credit_cw, inc=1,
                    device_id=(left,), device_id_type=pl.DeviceIdType.MESH,
                )
            add_stage(acc_cw, dst_slot, stage_cw, HALF, ch, track)
            pending["ccw", 1].wait()
            if s < N_DEV - 2:
                pl.semaphore_signal(
                    credit_ccw, inc=1,
                    device_id=(right,), device_id_type=pl.DeviceIdType.MESH,
                )
            add_stage(acc_ccw, dst_slot, stage_ccw, HALF, ch, track)

        amax_loc = functools.reduce(jnp.maximum, own_maxes)

        amax_ref[my] = jnp.full((8, 128), amax_loc, dtype=jnp.float32)
        amax_rdmas = []
        for t in range(N_DEV - 1):
            tgt = lax.rem(my + 1 + t, N_DEV)
            rdma = pltpu.make_async_remote_copy(
                src_ref=amax_ref.at[my],
                dst_ref=amax_ref.at[my],
                send_sem=amax_send_sems.at[t],
                recv_sem=amax_recv_sems.at[t],
                device_id=(tgt,),
                device_id_type=pl.DeviceIdType.MESH,
            )
            rdma.start()
            amax_rdmas.append(rdma)
        for rdma in amax_rdmas:
            rdma.wait()
        amax = jnp.max(amax_ref[...])

        scale = amax / 127.0
        inv_scale = 1.0 / scale

        def quant_half(acc, c_own, col0):
            for r0 in range(0, ch, SUB):
                yq = jnp.clip(
                    jnp.round(
                        acc[0, pl.ds(r0, SUB), :].astype(jnp.float32)
                        * inv_scale
                    ),
                    -127.0, 127.0,
                )
                q_ref[pl.ds(c_own * ch + r0, SUB), pl.ds(col0, nh)] = (
                    yq.astype(jnp.int8)
                )

        deq_pending = {}

        def dequant_half(c, col0):
            for r0 in range(0, ch, SUB):
                slot = (r0 // SUB) % 2
                if slot in deq_pending:
                    deq_pending.pop(slot).wait()
                sl = pl.ds(c * ch + r0, SUB)
                cs = pl.ds(col0, nh)
                deq_tiles[slot] = (
                    q_ref[sl, cs].astype(jnp.float32) * scale
                ).astype(jnp.bfloat16)
                copy = pltpu.make_async_copy(
                    deq_tiles.at[slot],
                    out_ref.at[sl, cs],
                    deq_sems.at[slot],
                )
                copy.start()
                deq_pending[slot] = copy

        def ag_rdma(c, col0, s_sems, r_sems, t, tgt):
            return pltpu.make_async_remote_copy(
                src_ref=q_ref.at[pl.ds(c * ch, ch), pl.ds(col0, nh)],
                dst_ref=q_ref.at[pl.ds(c * ch, ch), pl.ds(col0, nh)],
                send_sem=s_sems.at[2 * (N_DEV - 1) + t],
                recv_sem=r_sems.at[2 * (N_DEV - 1) + t],
                device_id=(tgt,),
                device_id_type=pl.DeviceIdType.MESH,
            )

        for t in range(N_DEV - 1):
            c_cw = lax.rem(my + 1 - t + N_DEV, N_DEV)
            c_ccw = lax.rem(my - 1 + t + N_DEV, N_DEV)
            if t == 0:
                quant_half(acc_cw, c_own_cw, 0)
                rdma_cw = ag_rdma(c_cw, 0, send_cw, recv_cw, t, right)
                rdma_cw.start()
                quant_half(acc_ccw, c_own_ccw, nh)
                rdma_ccw = ag_rdma(c_ccw, nh, send_ccw, recv_ccw, t, left)
                rdma_ccw.start()
            else:
                rdma_cw = ag_rdma(c_cw, 0, send_cw, recv_cw, t, right)
                rdma_ccw = ag_rdma(c_ccw, nh, send_ccw, recv_ccw, t, left)
                rdma_cw.start()
                rdma_ccw.start()
            dequant_half(c_cw, 0)
            dequant_half(c_ccw, nh)
            rdma_cw.wait()
            rdma_ccw.wait()

        dequant_half(lax.rem(my + 2, N_DEV), 0)
        dequant_half(lax.rem(my + 2, N_DEV), nh)
        for copy in deq_pending.values():
            copy.wait()

    n_sems = 2 * (N_DEV - 1) + (N_DEV - 1)
    return pl.pallas_call(
        body,
        out_shape=jax.ShapeDtypeStruct((m, n), jnp.bfloat16),
        in_specs=[
            pl.BlockSpec(memory_space=pltpu.VMEM),
            pl.BlockSpec(memory_space=pltpu.VMEM),
        ],
        out_specs=pl.BlockSpec(memory_space=pl.ANY),
        scratch_shapes=[
            pltpu.VMEM((2, ch, n // 2), jnp.bfloat16),
            pltpu.VMEM((2, ch, n // 2), jnp.bfloat16),
            pltpu.VMEM((m, n), jnp.int8),
            pltpu.VMEM((ch, n // 2), jnp.bfloat16),
            pltpu.VMEM((ch, n // 2), jnp.bfloat16),
            pltpu.VMEM((2, 256, n // 2), jnp.bfloat16),
            pltpu.VMEM((N_DEV, 8, 128), jnp.float32),
            pltpu.SemaphoreType.DMA((n_sems,)),
            pltpu.SemaphoreType.DMA((n_sems,)),
            pltpu.SemaphoreType.DMA((n_sems,)),
            pltpu.SemaphoreType.DMA((n_sems,)),
            pltpu.SemaphoreType.DMA((N_DEV - 1,)),
            pltpu.SemaphoreType.DMA((N_DEV - 1,)),
            pltpu.SemaphoreType.DMA((2,)),
            pltpu.SemaphoreType.REGULAR,
            pltpu.SemaphoreType.REGULAR,
        ],
        compiler_params=pltpu.CompilerParams(
            collective_id=0,
            vmem_limit_bytes=40 * 1024 * 1024,
        ),
    )(x, w_mat)


# device time: 148831 ns/iter; 3.2128x vs baseline; 1.0274x over previous
import functools

import jax
import jax.numpy as jnp
from jax import lax
from jax.experimental import pallas as pl
from jax.experimental.pallas import tpu as pltpu

N_DEV = 4


def kernel(x, w_mat):
    m, k_per = x.shape
    _, n = w_mat.shape
    ch = m // N_DEV
    nh = n // 2

    w_mat = w_mat.astype(jnp.bfloat16)

    def body(x_ref, w_ref, out_ref, acc_cw, acc_ccw, q_ref, stage_cw,
             stage_ccw, deq_tiles, amax_ref, send_cw, recv_cw, send_ccw,
             recv_ccw, amax_send_sems, amax_recv_sems, deq_sems,
             credit_cw, credit_ccw):
        my = lax.axis_index("i")
        left = lax.rem(my + N_DEV - 1, N_DEV)
        right = lax.rem(my + 1, N_DEV)

        barrier_sem = pltpu.get_barrier_semaphore()
        for nbr in (left, right):
            pl.semaphore_signal(
                barrier_sem, inc=1,
                device_id=(nbr,), device_id_type=pl.DeviceIdType.MESH,
            )
        pl.semaphore_wait(barrier_sem, 2)

        SUB = 256
        HALF = ch // 2

        def gemm_into(dst, c, col0, r_lo, r_hi):
            for r0 in range(r_lo, r_hi, SUB):
                p = jnp.dot(
                    x_ref[pl.ds(c * ch + r0, SUB), :].astype(jnp.bfloat16),
                    w_ref[:, pl.ds(col0, nh)],
                    preferred_element_type=jnp.float32,
                )
                dst[pl.ds(r0, SUB), :] = p.astype(jnp.bfloat16)

        own_maxes = []

        def add_stage(acc, slot, stage, r_lo, r_hi, track):
            for r0 in range(r_lo, r_hi, SUB):
                sl = pl.ds(r0, SUB)
                v = (
                    acc[slot, sl, :].astype(jnp.float32)
                    + stage[sl, :].astype(jnp.float32)
                )
                acc[slot, sl, :] = v.astype(jnp.bfloat16)
                if track:
                    own_maxes.append(jnp.max(jnp.abs(v)))

        def rs_rdma(acc, src_slot, dst_slot, s_sems, r_sems, idx, half,
                    tgt):
            return pltpu.make_async_remote_copy(
                src_ref=acc.at[src_slot, pl.ds(half * HALF, HALF), :],
                dst_ref=acc.at[dst_slot, pl.ds(half * HALF, HALF), :],
                send_sem=s_sems.at[idx],
                recv_sem=r_sems.at[idx],
                device_id=(tgt,),
                device_id_type=pl.DeviceIdType.MESH,
            )

        c_own_cw = lax.rem(my + 1, N_DEV)
        c_own_ccw = lax.rem(my + N_DEV - 1, N_DEV)

        pending = {}
        gemm_into(acc_cw.at[1], my, 0, 0, HALF)
        gemm_into(acc_ccw.at[1], my, nh, 0, HALF)
        for d, acc, s_sems, r_sems, tgt in (
            ("cw", acc_cw, send_cw, recv_cw, right),
            ("ccw", acc_ccw, send_ccw, recv_ccw, left),
        ):
            r = rs_rdma(acc, 1, 0, s_sems, r_sems, 0, 0, tgt)
            r.start()
            pending[d, 0] = r
        gemm_into(acc_cw.at[1], my, 0, HALF, ch)
        gemm_into(acc_ccw.at[1], my, nh, HALF, ch)
        for d, acc, s_sems, r_sems, tgt in (
            ("cw", acc_cw, send_cw, recv_cw, right),
            ("ccw", acc_ccw, send_ccw, recv_ccw, left),
        ):
            r = rs_rdma(acc, 1, 0, s_sems, r_sems, 1, 1, tgt)
            r.start()
            pending[d, 1] = r

        for s in range(N_DEV - 1):
            src_slot = (s + 1) % 2
            dst_slot = s % 2
            if s >= 1:
                pl.semaphore_wait(credit_cw, 1)
                pl.semaphore_wait(credit_ccw, 1)
                for sub in (0, 1):
                    r = rs_rdma(acc_cw, src_slot, dst_slot, send_cw,
                                recv_cw, 2 * s + sub, sub, right)
                    r.start()
                    pending["cw", sub] = r
                    r = rs_rdma(acc_ccw, src_slot, dst_slot, send_ccw,
                                recv_ccw, 2 * s + sub, sub, left)
                    r.start()
                    pending["ccw", sub] = r
            c_recv_cw = lax.rem(my - s - 1 + N_DEV, N_DEV)
            c_recv_ccw = lax.rem(my + s + 1, N_DEV)
            gemm_into(stage_cw, c_recv_cw, 0, 0, ch)
            gemm_into(stage_ccw, c_recv_ccw, nh, 0, ch)
            track = s == N_DEV - 2
            pending["cw", 0].wait()
            add_stage(acc_cw, dst_slot, stage_cw, 0, HALF, track)
            pending["ccw", 0].wait()
            add_stage(acc_ccw, dst_slot, stage_ccw, 0, HALF, track)
            pending["cw", 1].wait()
            if s < N_DEV - 2:
                pl.semaphore_signal(
                    credit_cw, inc=1,
                    device_id=(left,), device_id_type=pl.DeviceIdType.MESH,
                )
            add_stage(acc_cw, dst_slot, stage_cw, HALF, ch, track)
            pending["ccw", 1].wait()
            if s < N_DEV - 2:
                pl.semaphore_signal(
                    credit_ccw, inc=1,
                    device_id=(right,), device_id_type=pl.DeviceIdType.MESH,
                )
            add_stage(acc_ccw, dst_slot, stage_ccw, HALF, ch, track)

        amax_loc = functools.reduce(jnp.maximum, own_maxes)

        amax_ref[my] = jnp.full((8, 128), amax_loc, dtype=jnp.float32)
        amax_rdmas = []
        for t in range(N_DEV - 1):
            tgt = lax.rem(my + 1 + t, N_DEV)
            rdma = pltpu.make_async_remote_copy(
                src_ref=amax_ref.at[my],
                dst_ref=amax_ref.at[my],
                send_sem=amax_send_sems.at[t],
                recv_sem=amax_recv_sems.at[t],
                device_id=(tgt,),
                device_id_type=pl.DeviceIdType.MESH,
            )
            rdma.start()
            amax_rdmas.append(rdma)
        for rdma in amax_rdmas:
            rdma.wait()
        amax = jnp.max(amax_ref[...])

        scale = amax / 127.0
        inv_scale = 1.0 / scale

        def quant_rows(acc, c_own, col0, r_lo, r_hi):
            for r0 in range(r_lo, r_hi, SUB):
                yq = jnp.clip(
                    jnp.round(
                        acc[0, pl.ds(r0, SUB), :].astype(jnp.float32)
                        * inv_scale
                    ),
                    -127.0, 127.0,
                )
                q_ref[pl.ds(c_own * ch + r0, SUB), pl.ds(col0, nh)] = (
                    yq.astype(jnp.int8)
                )

        deq_pending = {}

        def dequant_rows(c, col0, r_lo, r_hi):
            for r0 in range(r_lo, r_hi, SUB):
                slot = (r0 // SUB) % 2
                if slot in deq_pending:
                    deq_pending.pop(slot).wait()
                sl = pl.ds(c * ch + r0, SUB)
                cs = pl.ds(col0, nh)
                deq_tiles[slot] = (
                    q_ref[sl, cs].astype(jnp.float32) * scale
                ).astype(jnp.bfloat16)
                copy = pltpu.make_async_copy(
                    deq_tiles.at[slot],
                    out_ref.at[sl, cs],
                    deq_sems.at[slot],
                )
                copy.start()
                deq_pending[slot] = copy

        def ag_rdma(c, col0, s_sems, r_sems, t, sub, tgt):
            rows = pl.ds(c * ch + sub * HALF, HALF)
            return pltpu.make_async_remote_copy(
                src_ref=q_ref.at[rows, pl.ds(col0, nh)],
                dst_ref=q_ref.at[rows, pl.ds(col0, nh)],
                send_sem=s_sems.at[2 * (N_DEV - 1) + 2 * t + sub],
                recv_sem=r_sems.at[2 * (N_DEV - 1) + 2 * t + sub],
                device_id=(tgt,),
                device_id_type=pl.DeviceIdType.MESH,
            )

        for t in range(N_DEV - 1):
            c_cw = lax.rem(my + 1 - t + N_DEV, N_DEV)
            c_ccw = lax.rem(my - 1 + t + N_DEV, N_DEV)
            ag_pending = []
            for sub in (0, 1):
                if t == 0:
                    quant_rows(acc_cw, c_own_cw, 0,
                               sub * HALF, (sub + 1) * HALF)
                r = ag_rdma(c_cw, 0, send_cw, recv_cw, t, sub, right)
                r.start()
                ag_pending.append(r)
                if t == 0:
                    quant_rows(acc_ccw, c_own_ccw, nh,
                               sub * HALF, (sub + 1) * HALF)
                r = ag_rdma(c_ccw, nh, send_ccw, recv_ccw, t, sub, left)
                r.start()
                ag_pending.append(r)
            dequant_rows(c_cw, 0, 0, ch)
            dequant_rows(c_ccw, nh, 0, ch)
            if t < N_DEV - 2:
                for r in ag_pending:
                    r.wait()
            else:
                c_last = lax.rem(my + 2, N_DEV)
                ag_pending[0].wait()
                ag_pending[1].wait()
                dequant_rows(c_last, 0, 0, HALF)
                dequant_rows(c_last, nh, 0, HALF)
                ag_pending[2].wait()
                ag_pending[3].wait()
                dequant_rows(c_last, 0, HALF, ch)
                dequant_rows(c_last, nh, HALF, ch)

        for copy in deq_pending.values():
            copy.wait()

    n_sems = 2 * (N_DEV - 1) + 2 * (N_DEV - 1)
    return pl.pallas_call(
        body,
        out_shape=jax.ShapeDtypeStruct((m, n), jnp.bfloat16),
        in_specs=[
            pl.BlockSpec(memory_space=pltpu.VMEM),
            pl.BlockSpec(memory_space=pltpu.VMEM),
        ],
        out_specs=pl.BlockSpec(memory_space=pl.ANY),
        scratch_shapes=[
            pltpu.VMEM((2, ch, n // 2), jnp.bfloat16),
            pltpu.VMEM((2, ch, n // 2), jnp.bfloat16),
            pltpu.VMEM((m, n), jnp.int8),
            pltpu.VMEM((ch, n // 2), jnp.bfloat16),
            pltpu.VMEM((ch, n // 2), jnp.bfloat16),
            pltpu.VMEM((2, 256, n // 2), jnp.bfloat16),
            pltpu.VMEM((N_DEV, 8, 128), jnp.float32),
            pltpu.SemaphoreType.DMA((n_sems,)),
            pltpu.SemaphoreType.DMA((n_sems,)),
            pltpu.SemaphoreType.DMA((n_sems,)),
            pltpu.SemaphoreType.DMA((n_sems,)),
            pltpu.SemaphoreType.DMA((N_DEV - 1,)),
            pltpu.SemaphoreType.DMA((N_DEV - 1,)),
            pltpu.SemaphoreType.DMA((2,)),
            pltpu.SemaphoreType.REGULAR,
            pltpu.SemaphoreType.REGULAR,
        ],
        compiler_params=pltpu.CompilerParams(
            collective_id=0,
            vmem_limit_bytes=40 * 1024 * 1024,
        ),
    )(x, w_mat)
